# baseline (speedup 1.0000x reference)
import functools

import jax
import jax.numpy as jnp
from jax import lax
from jax.experimental import pallas as pl
from jax.experimental.pallas import tpu as pltpu
from jax.experimental.pallas import tpu_sc as plsc

_GBLOCK = 120
_GPW = 1
_WBLOCK = _GPW * _GBLOCK
_NBUF = 2


@functools.lru_cache(maxsize=None)
def _build(n_nodes: int, n_vocab: int, dim: int):
    info = plsc.get_sparse_core_info()
    nc, ns = info.num_cores, info.num_subcores
    nw = nc * ns

    npw = n_nodes // (nw * _WBLOCK)
    span = npw * _WBLOCK
    tail = n_nodes - nw * span
    assert span % 8 == 0 and tail % 8 == 0
    tail_sub = 8
    n_tail_workers = tail // tail_sub if tail else 0
    assert n_tail_workers <= nw and (tail == 0 or tail % tail_sub == 0)

    mesh = plsc.VectorSubcoreMesh(core_axis_name="c", subcore_axis_name="s")

    @functools.partial(
        pl.kernel,
        out_type=jax.ShapeDtypeStruct((n_nodes, dim), jnp.float32),
        mesh=mesh,
        scratch_types=[
            pltpu.VMEM_SHARED((ns, n_vocab, dim), jnp.float32),
            pltpu.VMEM((span,), jnp.int32),
            pltpu.VMEM((_NBUF, _WBLOCK, dim), jnp.float32),
            pltpu.VMEM((tail_sub,), jnp.int32),
            pltpu.VMEM((tail_sub, dim), jnp.float32),
            pltpu.SemaphoreType.DMA,
            pltpu.SemaphoreType.DMA,
            pltpu.SemaphoreType.DMA,
            pltpu.SemaphoreType.DMA,
        ],
    )
    def emb_kernel(x_hbm, w_hbm, out_hbm, table_s, idx_v, rows_v, tidx_v,
                   trow_v, sem_g, sem_w0, sem_w1, sem_tw):
        sid = lax.axis_index("s")
        wid = sid * nc + lax.axis_index("c")
        base = wid * span

        pltpu.sync_copy(w_hbm, table_s.at[sid])

        pltpu.sync_copy(x_hbm.at[pl.ds(base, span)], idx_v)
        if tail:
            @pl.when(wid >= nw - n_tail_workers)
            def _():
                k = wid - (nw - n_tail_workers)
                tbase = nw * span + k * tail_sub
                pltpu.sync_copy(x_hbm.at[pl.ds(tbase, tail_sub)], tidx_v)

        plsc.subcore_barrier()

        w_sems = (sem_w0, sem_w1)

        def out_slot(u):
            return out_hbm.at[pl.ds(base + u * _WBLOCK, _WBLOCK)]

        def gather_into(u, b):
            for g in range(_GPW):
                pltpu.async_copy(
                    table_s.at[sid].at[idx_v.at[pl.ds(u * _WBLOCK
                                                      + g * _GBLOCK,
                                                      _GBLOCK)]],
                    rows_v.at[b, pl.ds(g * _GBLOCK, _GBLOCK)],
                    sem_g).wait()

        def loop_body(p, carry):
            for b in range(_NBUF):
                u = p * _NBUF + b

                @pl.when(p >= 1)
                def _():
                    pltpu.make_async_copy(rows_v.at[b], out_slot(u),
                                          w_sems[b]).wait()

                gather_into(u, b)
                pltpu.async_copy(rows_v.at[b], out_slot(u), w_sems[b])
            return carry

        nloop = (npw // _NBUF) * _NBUF
        lax.fori_loop(0, npw // _NBUF, loop_body, 0)

        for r in range(npw - nloop):
            u = nloop + r
            pltpu.make_async_copy(rows_v.at[r], out_slot(u),
                                  w_sems[r]).wait()
            gather_into(u, r)
            pltpu.async_copy(rows_v.at[r], out_slot(u), w_sems[r])

        if tail:
            @pl.when(wid >= nw - n_tail_workers)
            def _():
                k = wid - (nw - n_tail_workers)
                tbase = nw * span + k * tail_sub
                pltpu.async_copy(table_s.at[sid].at[tidx_v], trow_v,
                                 sem_g).wait()
                pltpu.async_copy(trow_v, out_hbm.at[pl.ds(tbase, tail_sub)],
                                 sem_tw)

        nrem = npw - nloop
        last = [nloop + r for r in range(nrem)] + \
               [nloop - _NBUF + b for b in range(nrem, _NBUF)]
        for b in range(_NBUF):
            pltpu.make_async_copy(rows_v.at[b], out_slot(last[b]),
                                  w_sems[b]).wait()
        if tail:
            @pl.when(wid >= nw - n_tail_workers)
            def _():
                k = wid - (nw - n_tail_workers)
                tbase = nw * span + k * tail_sub
                pltpu.make_async_copy(trow_v,
                                      out_hbm.at[pl.ds(tbase, tail_sub)],
                                      sem_tw).wait()

    return emb_kernel


def kernel(x, weight):
    n_nodes = x.shape[0]
    n_vocab, dim = weight.shape
    emb = _build(n_nodes, n_vocab, dim)
    return emb(x.astype(jnp.int32), weight)


# speedup vs baseline: 1.0358x; 1.0358x over previous
import functools

import jax
import jax.numpy as jnp
from jax import lax
from jax.experimental import pallas as pl
from jax.experimental.pallas import tpu as pltpu
from jax.experimental.pallas import tpu_sc as plsc

_GBLOCK = 120
_GPW = 1
_WBLOCK = _GPW * _GBLOCK
_NBUF = 2


@functools.lru_cache(maxsize=None)
def _build(n_nodes: int, n_vocab: int, dim: int):
    info = plsc.get_sparse_core_info()
    nc, ns = info.num_cores, info.num_subcores
    nw = nc * ns

    npw = n_nodes // (nw * _WBLOCK)
    span = npw * _WBLOCK
    tail = n_nodes - nw * span
    assert span % 8 == 0 and tail % 8 == 0
    tail_sub = 8
    n_tail_workers = tail // tail_sub if tail else 0
    assert n_tail_workers <= nw and (tail == 0 or tail % tail_sub == 0)

    mesh = plsc.VectorSubcoreMesh(core_axis_name="c", subcore_axis_name="s")

    @functools.partial(
        pl.kernel,
        out_type=jax.ShapeDtypeStruct((n_nodes, dim), jnp.float32),
        mesh=mesh,
        scratch_types=[
            pltpu.VMEM_SHARED((n_vocab, dim), jnp.float32),
            pltpu.VMEM((span,), jnp.int32),
            pltpu.VMEM((_NBUF, _WBLOCK, dim), jnp.float32),
            pltpu.VMEM((tail_sub,), jnp.int32),
            pltpu.VMEM((tail_sub, dim), jnp.float32),
            pltpu.SemaphoreType.DMA,
            pltpu.SemaphoreType.DMA,
            pltpu.SemaphoreType.DMA,
            pltpu.SemaphoreType.DMA,
        ],
    )
    def emb_kernel(x_hbm, w_hbm, out_hbm, table_s, idx_v, rows_v, tidx_v,
                   trow_v, sem_g, sem_w0, sem_w1, sem_tw):
        sid = lax.axis_index("s")
        wid = sid * nc + lax.axis_index("c")
        base = wid * span

        @pl.when(sid == 0)
        def _():
            pltpu.sync_copy(w_hbm, table_s)

        idx_dma = pltpu.make_async_copy(x_hbm.at[pl.ds(base, span)], idx_v,
                                        sem_tw)
        idx_dma.start()
        if tail:
            @pl.when(wid >= nw - n_tail_workers)
            def _():
                k = wid - (nw - n_tail_workers)
                tbase = nw * span + k * tail_sub
                pltpu.sync_copy(x_hbm.at[pl.ds(tbase, tail_sub)], tidx_v)

        plsc.subcore_barrier()
        idx_dma.wait()

        w_sems = (sem_w0, sem_w1)

        def out_slot(u):
            return out_hbm.at[pl.ds(base + u * _WBLOCK, _WBLOCK)]

        def gather_into(u, b):
            for g in range(_GPW):
                pltpu.async_copy(
                    table_s.at[idx_v.at[pl.ds(u * _WBLOCK + g * _GBLOCK,
                                              _GBLOCK)]],
                    rows_v.at[b, pl.ds(g * _GBLOCK, _GBLOCK)],
                    sem_g).wait()

        def loop_body(p, carry):
            for b in range(_NBUF):
                u = p * _NBUF + b

                @pl.when(p >= 1)
                def _():
                    pltpu.make_async_copy(rows_v.at[b], out_slot(u),
                                          w_sems[b]).wait()

                gather_into(u, b)
                pltpu.async_copy(rows_v.at[b], out_slot(u), w_sems[b])
            return carry

        nloop = (npw // _NBUF) * _NBUF
        lax.fori_loop(0, npw // _NBUF, loop_body, 0)

        for r in range(npw - nloop):
            u = nloop + r
            pltpu.make_async_copy(rows_v.at[r], out_slot(u),
                                  w_sems[r]).wait()
            gather_into(u, r)
            pltpu.async_copy(rows_v.at[r], out_slot(u), w_sems[r])

        if tail:
            @pl.when(wid >= nw - n_tail_workers)
            def _():
                k = wid - (nw - n_tail_workers)
                tbase = nw * span + k * tail_sub
                pltpu.async_copy(table_s.at[tidx_v], trow_v, sem_g).wait()
                pltpu.async_copy(trow_v, out_hbm.at[pl.ds(tbase, tail_sub)],
                                 sem_tw)

        nrem = npw - nloop
        last = [nloop + r for r in range(nrem)] + \
               [nloop - _NBUF + b for b in range(nrem, _NBUF)]
        for b in range(_NBUF):
            pltpu.make_async_copy(rows_v.at[b], out_slot(last[b]),
                                  w_sems[b]).wait()
        if tail:
            @pl.when(wid >= nw - n_tail_workers)
            def _():
                k = wid - (nw - n_tail_workers)
                tbase = nw * span + k * tail_sub
                pltpu.make_async_copy(trow_v,
                                      out_hbm.at[pl.ds(tbase, tail_sub)],
                                      sem_tw).wait()

    return emb_kernel


def kernel(x, weight):
    n_nodes = x.shape[0]
    n_vocab, dim = weight.shape
    emb = _build(n_nodes, n_vocab, dim)
    return emb(x.astype(jnp.int32), weight)
